# Initial kernel scaffold; baseline (speedup 1.0000x reference)
import jax
import jax.numpy as jnp
from jax.experimental import pallas as pl


def kernel(x, lengths, table, W1, b1, W2, b2, W3, b3):
    raise NotImplementedError("write your pallas kernel here")


# speedup vs baseline: 2.3083x; 2.3083x over previous
import functools

import jax
import jax.numpy as jnp
from jax import lax
from jax.experimental import pallas as pl
from jax.experimental.pallas import tpu as pltpu
from jax.experimental.pallas import tpu_sc as plsc

VOCAB = 1000000
DIM = 64
B = 16384
L = 50
H1 = 128
H2 = 64
OUT = 10

NC = 2
NS = 16
NW = NC * NS
BPW = B // NW
CB = 8
NCHUNK = BPW // CB
IDX_PER_CHUNK = CB * L
GI = 80
G = IDX_PER_CHUNK // GI
NLG = DIM // 16


def _pool_body(xf_hbm, table_hbm, out_hbm, idx_v, rows_v, acc_v, sem):
    w = lax.axis_index("s") * NC + lax.axis_index("c")

    def chunk(j, carry):
        xoff = w * (BPW * L) + j * IDX_PER_CHUNK
        pltpu.sync_copy(xf_hbm.at[pl.ds(xoff, IDX_PER_CHUNK)], idx_v)
        cps = [
            pltpu.async_copy(
                table_hbm.at[idx_v.at[pl.ds(t * GI, GI)]],
                rows_v.at[pl.ds(t * GI, GI), :],
                sem,
            )
            for t in range(G)
        ]
        for cp in cps:
            cp.wait()

        for b in range(CB):
            def red(k, accs):
                r = b * L + k
                return tuple(
                    accs[g] + rows_v[r, pl.ds(g * 16, 16)] for g in range(NLG)
                )

            accs = lax.fori_loop(
                0, L, red,
                tuple(jnp.zeros((16,), jnp.float32) for _ in range(NLG)),
            )
            for g in range(NLG):
                acc_v[b, pl.ds(g * 16, 16)] = accs[g]

        pltpu.sync_copy(acc_v, out_hbm.at[pl.ds(w * BPW + j * CB, CB), :])
        return carry

    lax.fori_loop(0, NCHUNK, chunk, 0)


def _make_pool():
    mesh = plsc.VectorSubcoreMesh(core_axis_name="c", subcore_axis_name="s")
    return pl.kernel(
        _pool_body,
        mesh=mesh,
        out_type=jax.ShapeDtypeStruct((B, DIM), jnp.float32),
        scratch_types=[
            pltpu.VMEM((IDX_PER_CHUNK,), jnp.int32),
            pltpu.VMEM((IDX_PER_CHUNK, DIM), jnp.float32),
            pltpu.VMEM((CB, DIM), jnp.float32),
            pltpu.SemaphoreType.DMA,
        ],
        compiler_params=pltpu.CompilerParams(use_tc_tiling_on_sc=False),
    )


def _mlp_kernel(sums_ref, len_ref, w1_ref, b1_ref, w2_ref, b2_ref, w3_ref,
                b3_ref, out_ref):
    rep = sums_ref[...] / len_ref[...]
    h1 = jnp.tanh(
        jnp.dot(rep, w1_ref[...], preferred_element_type=jnp.float32)
        + b1_ref[...]
    )
    z2 = (
        jnp.dot(h1, w2_ref[...], preferred_element_type=jnp.float32)
        + b2_ref[...]
    )
    h2 = jnp.where(z2 >= 0, z2, 0.01 * z2)
    out_ref[...] = (
        jnp.dot(h2, w3_ref[...], preferred_element_type=jnp.float32)
        + b3_ref[...]
    )


def _mlp(sums, lens_f, W1, b1, W2, b2, W3, b3):
    bs = 2048
    grid = B // bs
    full = lambda shape: pl.BlockSpec(shape, lambda i: (0, 0))
    return pl.pallas_call(
        _mlp_kernel,
        grid=(grid,),
        in_specs=[
            pl.BlockSpec((bs, DIM), lambda i: (i, 0)),
            pl.BlockSpec((bs, 1), lambda i: (i, 0)),
            full((DIM, H1)),
            full((1, H1)),
            full((H1, H2)),
            full((1, H2)),
            full((H2, OUT)),
            full((1, OUT)),
        ],
        out_specs=pl.BlockSpec((bs, OUT), lambda i: (i, 0)),
        out_shape=jax.ShapeDtypeStruct((B, OUT), jnp.float32),
    )(sums, lens_f, W1, b1, W2, b2, W3, b3)


@jax.jit
def kernel(x, lengths, table, W1, b1, W2, b2, W3, b3):
    xf = x.reshape(B * L)
    sums = _make_pool()(xf, table)
    lens_f = lengths.astype(jnp.float32).reshape(B, 1)
    return _mlp(
        sums, lens_f, W1,
        b1.reshape(1, H1), W2, b2.reshape(1, H2), W3, b3.reshape(1, OUT),
    )


# speedup vs baseline: 2.7094x; 1.1738x over previous
import functools

import jax
import jax.numpy as jnp
from jax import lax
from jax.experimental import pallas as pl
from jax.experimental.pallas import tpu as pltpu
from jax.experimental.pallas import tpu_sc as plsc

VOCAB = 1000000
DIM = 64
B = 16384
L = 50
H1 = 128
H2 = 64
OUT = 10

NC = 2
NS = 16
NW = NC * NS
BPW = B // NW
CB = 16
NCHUNK = BPW // CB
IDX_PER_CHUNK = CB * L
GI = 80
G = IDX_PER_CHUNK // GI
NLG = DIM // 16
KU = 10


def _pool_body(xf_hbm, table_hbm, out_hbm,
               idx0, idx1, rows0, rows1, acc0, acc1,
               isem, gsem0, gsem1, osem0, osem1):
    w = lax.axis_index("s") * NC + lax.axis_index("c")
    xbase = w * (BPW * L)
    obase = w * BPW
    idx_v = (idx0, idx1)
    rows_v = (rows0, rows1)
    acc_v = (acc0, acc1)
    gsem = (gsem0, gsem1)
    osem = (osem0, osem1)

    def issue_idx(c, p):
        pltpu.async_copy(
            xf_hbm.at[pl.ds(xbase + c * IDX_PER_CHUNK, IDX_PER_CHUNK)],
            idx_v[p], isem)

    def wait_idx(p):
        pltpu.make_async_copy(
            xf_hbm.at[pl.ds(0, IDX_PER_CHUNK)], idx_v[p], isem).wait()

    def issue_gathers(p):
        for t in range(G):
            pltpu.async_copy(
                table_hbm.at[idx_v[p].at[pl.ds(t * GI, GI)]],
                rows_v[p].at[pl.ds(t * GI, GI), :], gsem[p])

    def drain_gathers(p):
        for t in range(G):
            pltpu.make_async_copy(
                table_hbm.at[idx_v[p].at[pl.ds(t * GI, GI)]],
                rows_v[p].at[pl.ds(t * GI, GI), :], gsem[p]).wait()

    def issue_out(c, p):
        pltpu.async_copy(
            acc_v[p], out_hbm.at[pl.ds(obase + c * CB, CB), :], osem[p])

    def drain_out(p):
        pltpu.make_async_copy(
            acc_v[p], out_hbm.at[pl.ds(0, CB), :], osem[p]).wait()

    def reduce(p):
        rv = rows_v[p]
        for b in range(CB):
            def red(kk, accs):
                base = b * L + kk * KU
                for u in range(KU):
                    accs = tuple(
                        accs[g] + rv[base + u, pl.ds(g * 16, 16)]
                        for g in range(NLG)
                    )
                return accs

            accs = lax.fori_loop(
                0, L // KU, red,
                tuple(jnp.zeros((16,), jnp.float32) for _ in range(NLG)),
            )
            for g in range(NLG):
                acc_v[p][b, pl.ds(g * 16, 16)] = accs[g]

    issue_idx(0, 0)
    wait_idx(0)
    issue_gathers(0)
    issue_idx(1, 1)

    def body(jj, carry):
        for p in (0, 1):
            j = 2 * jj + p

            @pl.when(j + 1 < NCHUNK)
            def _():
                wait_idx(1 - p)
                issue_gathers(1 - p)

            drain_gathers(p)

            @pl.when(j + 2 < NCHUNK)
            def _():
                issue_idx(j + 2, p)

            @pl.when(jj >= 1)
            def _():
                drain_out(p)

            reduce(p)
            issue_out(j, p)
        return carry

    lax.fori_loop(0, NCHUNK // 2, body, 0)
    drain_out(0)
    drain_out(1)


def _make_pool():
    mesh = plsc.VectorSubcoreMesh(core_axis_name="c", subcore_axis_name="s")
    return pl.kernel(
        _pool_body,
        mesh=mesh,
        out_type=jax.ShapeDtypeStruct((B, DIM), jnp.float32),
        scratch_types=[
            pltpu.VMEM((IDX_PER_CHUNK,), jnp.int32),
            pltpu.VMEM((IDX_PER_CHUNK,), jnp.int32),
            pltpu.VMEM((IDX_PER_CHUNK, DIM), jnp.float32),
            pltpu.VMEM((IDX_PER_CHUNK, DIM), jnp.float32),
            pltpu.VMEM((CB, DIM), jnp.float32),
            pltpu.VMEM((CB, DIM), jnp.float32),
            pltpu.SemaphoreType.DMA,
            pltpu.SemaphoreType.DMA,
            pltpu.SemaphoreType.DMA,
            pltpu.SemaphoreType.DMA,
            pltpu.SemaphoreType.DMA,
        ],
        compiler_params=pltpu.CompilerParams(use_tc_tiling_on_sc=False),
    )


def _mlp_kernel(sums_ref, len_ref, w1_ref, b1_ref, w2_ref, b2_ref, w3_ref,
                b3_ref, out_ref):
    rep = sums_ref[...] / len_ref[...]
    h1 = jnp.tanh(
        jnp.dot(rep, w1_ref[...], preferred_element_type=jnp.float32)
        + b1_ref[...]
    )
    z2 = (
        jnp.dot(h1, w2_ref[...], preferred_element_type=jnp.float32)
        + b2_ref[...]
    )
    h2 = jnp.where(z2 >= 0, z2, 0.01 * z2)
    out_ref[...] = (
        jnp.dot(h2, w3_ref[...], preferred_element_type=jnp.float32)
        + b3_ref[...]
    )


def _mlp(sums, lens_f, W1, b1, W2, b2, W3, b3):
    bs = 2048
    grid = B // bs
    full = lambda shape: pl.BlockSpec(shape, lambda i: (0, 0))
    return pl.pallas_call(
        _mlp_kernel,
        grid=(grid,),
        in_specs=[
            pl.BlockSpec((bs, DIM), lambda i: (i, 0)),
            pl.BlockSpec((bs, 1), lambda i: (i, 0)),
            full((DIM, H1)),
            full((1, H1)),
            full((H1, H2)),
            full((1, H2)),
            full((H2, OUT)),
            full((1, OUT)),
        ],
        out_specs=pl.BlockSpec((bs, OUT), lambda i: (i, 0)),
        out_shape=jax.ShapeDtypeStruct((B, OUT), jnp.float32),
    )(sums, lens_f, W1, b1, W2, b2, W3, b3)


@jax.jit
def kernel(x, lengths, table, W1, b1, W2, b2, W3, b3):
    xf = x.reshape(B * L)
    sums = _make_pool()(xf, table)
    lens_f = lengths.astype(jnp.float32).reshape(B, 1)
    return _mlp(
        sums, lens_f, W1,
        b1.reshape(1, H1), W2, b2.reshape(1, H2), W3, b3.reshape(1, OUT),
    )


# speedup vs baseline: 2.9859x; 1.1021x over previous
import functools

import jax
import jax.numpy as jnp
from jax import lax
from jax.experimental import pallas as pl
from jax.experimental.pallas import tpu as pltpu
from jax.experimental.pallas import tpu_sc as plsc

VOCAB = 1000000
DIM = 64
B = 16384
L = 50
H1 = 128
H2 = 64
OUT = 10

NC = 2
NS = 16
NW = NC * NS
BPW = B // NW
CB = 16
NCHUNK = BPW // CB
IDX_PER_CHUNK = CB * L
GI = 80
G = IDX_PER_CHUNK // GI
NLG = DIM // 16
KU = 10


def _pool_body(xf_hbm, table_hbm, out_hbm,
               idx0, idx1, rows0, rows1, acc0, acc1,
               isem, gsem0, gsem1, osem0, osem1):
    w = lax.axis_index("s") * NC + lax.axis_index("c")
    xbase = w * (BPW * L)
    obase = w * BPW
    idx_v = (idx0, idx1)
    rows_v = (rows0, rows1)
    acc_v = (acc0, acc1)
    gsem = (gsem0, gsem1)
    osem = (osem0, osem1)

    def issue_idx(c, p):
        pltpu.async_copy(
            xf_hbm.at[pl.ds(xbase + c * IDX_PER_CHUNK, IDX_PER_CHUNK)],
            idx_v[p], isem)

    def wait_idx(p):
        pltpu.make_async_copy(
            xf_hbm.at[pl.ds(0, IDX_PER_CHUNK)], idx_v[p], isem).wait()

    def issue_gathers(p):
        for t in range(G):
            pltpu.async_copy(
                table_hbm.at[idx_v[p].at[pl.ds(t * GI, GI)]],
                rows_v[p].at[pl.ds(t * GI, GI), :], gsem[p])

    def drain_gathers(p):
        for t in range(G):
            pltpu.make_async_copy(
                table_hbm.at[idx_v[p].at[pl.ds(t * GI, GI)]],
                rows_v[p].at[pl.ds(t * GI, GI), :], gsem[p]).wait()

    def issue_out(c, p):
        pltpu.async_copy(
            acc_v[p], out_hbm.at[pl.ds(obase + c * CB, CB), :], osem[p])

    def drain_out(p):
        pltpu.make_async_copy(
            acc_v[p], out_hbm.at[pl.ds(0, CB), :], osem[p]).wait()

    def reduce(p):
        rv = rows_v[p]
        for b in range(CB):
            def red(kk, accs):
                base = b * L + kk * KU
                for u in range(KU):
                    accs = tuple(
                        accs[g] + rv[base + u, pl.ds(g * 16, 16)]
                        for g in range(NLG)
                    )
                return accs

            accs = lax.fori_loop(
                0, L // KU, red,
                tuple(jnp.zeros((16,), jnp.float32) for _ in range(NLG)),
            )
            for g in range(NLG):
                acc_v[p][b, pl.ds(g * 16, 16)] = accs[g]

    issue_idx(0, 0)
    wait_idx(0)
    issue_gathers(0)
    issue_idx(1, 1)

    def body(jj, carry):
        for p in (0, 1):
            j = 2 * jj + p

            @pl.when(j + 1 < NCHUNK)
            def _():
                wait_idx(1 - p)
                issue_gathers(1 - p)

            drain_gathers(p)

            @pl.when(j + 2 < NCHUNK)
            def _():
                issue_idx(j + 2, p)

            @pl.when(jj >= 1)
            def _():
                drain_out(p)

            reduce(p)
            issue_out(j, p)
        return carry

    lax.fori_loop(0, NCHUNK // 2, body, 0)
    drain_out(0)
    drain_out(1)


def _make_pool():
    mesh = plsc.VectorSubcoreMesh(core_axis_name="c", subcore_axis_name="s")
    return pl.kernel(
        _pool_body,
        mesh=mesh,
        out_type=jax.ShapeDtypeStruct((B, DIM), jnp.float32),
        scratch_types=[
            pltpu.VMEM((IDX_PER_CHUNK,), jnp.int32),
            pltpu.VMEM((IDX_PER_CHUNK,), jnp.int32),
            pltpu.VMEM((IDX_PER_CHUNK, DIM), jnp.float32),
            pltpu.VMEM((IDX_PER_CHUNK, DIM), jnp.float32),
            pltpu.VMEM((CB, DIM), jnp.float32),
            pltpu.VMEM((CB, DIM), jnp.float32),
            pltpu.SemaphoreType.DMA,
            pltpu.SemaphoreType.DMA,
            pltpu.SemaphoreType.DMA,
            pltpu.SemaphoreType.DMA,
            pltpu.SemaphoreType.DMA,
        ],
        compiler_params=pltpu.CompilerParams(use_tc_tiling_on_sc=False),
    )


def _relayout_kernel(t_ref, o_ref):
    t = t_ref[...]
    n = t.shape[0]
    v0 = t[:, 0:8, :]
    v1 = t[:, 8:16, :]
    ev = jax.lax.broadcasted_iota(jnp.int32, (n, 4, DIM), 1) * 2
    e = jnp.concatenate(
        [jnp.take_along_axis(v0, ev, axis=1),
         jnp.take_along_axis(v1, ev, axis=1)], axis=1)
    o = jnp.concatenate(
        [jnp.take_along_axis(v0, ev + 1, axis=1),
         jnp.take_along_axis(v1, ev + 1, axis=1)], axis=1)
    o_ref[...] = jnp.concatenate([e, o], axis=2)


def _relayout(table):
    n = 250
    grid = (VOCAB // 16) // n
    out = pl.pallas_call(
        _relayout_kernel,
        grid=(grid,),
        in_specs=[pl.BlockSpec((n, 16, DIM), lambda i: (i, 0, 0))],
        out_specs=pl.BlockSpec((n, 8, 2 * DIM), lambda i: (i, 0, 0)),
        out_shape=jax.ShapeDtypeStruct((VOCAB // 16, 8, 2 * DIM), jnp.float32),
    )(table.reshape(VOCAB // 16, 16, DIM))
    return out.reshape(VOCAB, DIM)


def _mlp_kernel(sums_ref, len_ref, w1_ref, b1_ref, w2_ref, b2_ref, w3_ref,
                b3_ref, out_ref):
    rep = sums_ref[...] / len_ref[...]
    h1 = jnp.tanh(
        jnp.dot(rep, w1_ref[...], preferred_element_type=jnp.float32)
        + b1_ref[...]
    )
    z2 = (
        jnp.dot(h1, w2_ref[...], preferred_element_type=jnp.float32)
        + b2_ref[...]
    )
    h2 = jnp.where(z2 >= 0, z2, 0.01 * z2)
    out_ref[...] = (
        jnp.dot(h2, w3_ref[...], preferred_element_type=jnp.float32)
        + b3_ref[...]
    )


def _mlp(sums, lens_f, W1, b1, W2, b2, W3, b3):
    bs = 2048
    grid = B // bs
    full = lambda shape: pl.BlockSpec(shape, lambda i: (0, 0))
    return pl.pallas_call(
        _mlp_kernel,
        grid=(grid,),
        in_specs=[
            pl.BlockSpec((bs, DIM), lambda i: (i, 0)),
            pl.BlockSpec((bs, 1), lambda i: (i, 0)),
            full((DIM, H1)),
            full((1, H1)),
            full((H1, H2)),
            full((1, H2)),
            full((H2, OUT)),
            full((1, OUT)),
        ],
        out_specs=pl.BlockSpec((bs, OUT), lambda i: (i, 0)),
        out_shape=jax.ShapeDtypeStruct((B, OUT), jnp.float32),
    )(sums, lens_f, W1, b1, W2, b2, W3, b3)


@jax.jit
def kernel(x, lengths, table, W1, b1, W2, b2, W3, b3):
    xf = x.reshape(B * L)
    sums = _make_pool()(xf, _relayout(table))
    lens_f = lengths.astype(jnp.float32).reshape(B, 1)
    return _mlp(
        sums, lens_f, W1,
        b1.reshape(1, H1), W2, b2.reshape(1, H2), W3, b3.reshape(1, OUT),
    )
